# baseline (speedup 1.0000x reference)
import functools

import jax
import jax.numpy as jnp
from jax import lax
from jax.experimental import pallas as pl
from jax.experimental.pallas import tpu as pltpu
from jax.experimental.pallas import tpu_sc as plsc

_B = 16384
_IN_DIM = 100
_N_BINS = 10
_EMB = 16
_HID = 64
_OUT = 64
_FPAD = 128
_RSTRIDE = 65
_TWORDS = 65536
_BT = 4096

_B_SC = 1024
_B_TC = _B - _B_SC
_BT_MAIN = 3072
_NW = 32
_BW = _B_SC // _NW
_CS = 32


def _prep_kernel(x_ref, embp_ref, w1s_ref, codes_ref, t2_ref, cmax_ref,
                 cmax_scr):
    i = pl.program_id(0)
    G = pl.num_programs(0) - 1

    @pl.when(i < G)
    def _colmax_phase():
        part = jnp.max(jnp.abs(x_ref[...]), axis=0, keepdims=True)

        @pl.when(i == 0)
        def _():
            cmax_scr[...] = part

        @pl.when(i > 0)
        def _():
            cmax_scr[...] = jnp.maximum(cmax_scr[...], part)

    @pl.when(i == G)
    def _fold_codes_phase():
        acc = embp_ref[:, :, 0:1] * w1s_ref[0]
        for d in range(1, _EMB):
            acc = acc + embp_ref[:, :, d:d + 1] * w1s_ref[d]
        t2_ref[...] = acc
        cmax_ref[...] = cmax_scr[...]
        x = x_ref[_BT - _B_SC:, :]
        d = cmax_scr[...]
        bins = jnp.clip(x / d * (_N_BINS / 2.0) + _N_BINS / 2.0,
                        0.0, _N_BINS - 1).astype(jnp.int32)
        f_iota = jax.lax.broadcasted_iota(jnp.int32, x.shape, 1)
        codes_ref[...] = (f_iota * _N_BINS + bins) * _RSTRIDE


def _main_kernel(x_ref, cmax_ref, t2_ref, b1_ref, w2t_ref, b2_ref, o_ref,
                 t2bf_scr):
    i = pl.program_id(0)

    @pl.when(i == 0)
    def _():
        t2bf_scr[...] = t2_ref[...].astype(jnp.bfloat16).reshape(
            _N_BINS // 2, 2 * _FPAD, _HID)

    x = x_ref[...]
    d = cmax_ref[...]
    bins = jnp.clip(x / d * (_N_BINS / 2.0) + _N_BINS / 2.0,
                    0.0, _N_BINS - 1).astype(jnp.int32)
    pad = jnp.full((x.shape[0], _FPAD - _IN_DIM), -1, jnp.int32)
    binp = jnp.concatenate([bins, pad], axis=1).astype(jnp.bfloat16)
    bin2 = jnp.concatenate([binp, binp], axis=1)
    lane2 = jax.lax.broadcasted_iota(jnp.int32, (1, 2 * _FPAD), 1)
    off = (lane2 >= _FPAD).astype(jnp.bfloat16)
    h = None
    for q in range(_N_BINS // 2):
        nvec = off + jnp.bfloat16(2 * q)
        mask = (bin2 == nvec).astype(jnp.bfloat16)
        dq = jax.lax.dot(mask, t2bf_scr[q],
                         preferred_element_type=jnp.float32)
        h = dq if h is None else h + dq
    h = jnp.maximum(h + b1_ref[...], 0.0)
    out = jax.lax.dot(h, w2t_ref[...], preferred_element_type=jnp.float32)
    o_ref[...] = out + b2_ref[...]


@functools.cache
def _build_sc_lookup():
    return functools.partial(
        pl.kernel,
        mesh=plsc.VectorSubcoreMesh(core_axis_name="c", subcore_axis_name="s"),
        compiler_params=pltpu.CompilerParams(needs_layout_passes=False),
        out_type=jax.ShapeDtypeStruct((_B_SC * _HID,), jnp.float32),
        scratch_types=[
            pltpu.VMEM((_CS * _IN_DIM,), jnp.int32),
            pltpu.VMEM((_TWORDS,), jnp.float32),
            pltpu.VMEM((_CS * _HID,), jnp.float32),
        ],
    )(_sc_lookup_body)


def _sc_call(codes_flat, table_flat):
    return _build_sc_lookup()(codes_flat, table_flat)


def _sc_lookup_body(codes_hbm, table_hbm, out_hbm, codes_v, table_v, h_v):
    wid = lax.axis_index("s") * 2 + lax.axis_index("c")
    lane = lax.iota(jnp.int32, 16)
    lane_c = lane * _IN_DIM
    lane_h = lane * _HID
    pltpu.sync_copy(table_hbm, table_v)

    def chunk_body(c, carry):
        row0 = wid * _BW + c * _CS
        pltpu.sync_copy(codes_hbm.at[pl.ds(row0 * _IN_DIM, _CS * _IN_DIM)],
                        codes_v)

        def g_body(g, carry2):
            for hhc in range(4):
                def f_body(f, accs):
                    bases = plsc.load_gather(
                        codes_v, [lane_c + (g * (16 * _IN_DIM) + f)])
                    return tuple(
                        accs[p] + plsc.load_gather(
                            table_v, [bases + (hhc * 16 + p)])
                        for p in range(16))

                accs = lax.fori_loop(
                    0, _IN_DIM, f_body,
                    tuple(jnp.zeros((16,), jnp.float32) for _ in range(16)))
                for p in range(16):
                    plsc.store_scatter(
                        h_v, [lane_h + (g * (16 * _HID) + hhc * 16 + p)],
                        accs[p])
            return carry2

        lax.fori_loop(0, _CS // 16, g_body, 0)
        pltpu.sync_copy(h_v, out_hbm.at[pl.ds(row0 * _HID, _CS * _HID)])
        return carry

    lax.fori_loop(0, _BW // _CS, chunk_body, 0)


def _tail_kernel(oalias_ref, h_ref, b1_ref, w2t_ref, b2_ref, o_ref):
    h = jnp.maximum(h_ref[...] + b1_ref[...], 0.0)
    out = jax.lax.dot(h, w2t_ref[...], preferred_element_type=jnp.float32)
    o_ref[...] = out + b2_ref[...]


def kernel(X, emb, W1, b1, W2, b2):
    B, IN = X.shape
    G = B // _BT

    embp = jnp.pad(jnp.transpose(emb, (1, 0, 2)),
                   ((0, 0), (0, _FPAD - _IN_DIM), (0, 0)))
    w1s = jnp.pad(W1.T.reshape(_IN_DIM, _EMB, _HID).transpose(1, 0, 2),
                  ((0, 0), (0, _FPAD - _IN_DIM), (0, 0)))

    codes, t2, cmax = pl.pallas_call(
        _prep_kernel,
        grid=(G + 1,),
        in_specs=[
            pl.BlockSpec((_BT, IN), lambda i: (jnp.minimum(i, 3), 0)),
            pl.BlockSpec((_N_BINS, _FPAD, _EMB), lambda i: (0, 0, 0)),
            pl.BlockSpec((_EMB, _FPAD, _HID), lambda i: (0, 0, 0)),
        ],
        out_specs=[
            pl.BlockSpec((_B_SC, IN), lambda i: (0, 0)),
            pl.BlockSpec((_N_BINS, _FPAD, _HID), lambda i: (0, 0, 0)),
            pl.BlockSpec((1, IN), lambda i: (0, 0)),
        ],
        out_shape=[
            jax.ShapeDtypeStruct((_B_SC, IN), jnp.int32),
            jax.ShapeDtypeStruct((_N_BINS, _FPAD, _HID), jnp.float32),
            jax.ShapeDtypeStruct((1, IN), jnp.float32),
        ],
        scratch_shapes=[pltpu.VMEM((1, IN), jnp.float32)],
    )(X, embp, w1s)

    out_buf = pl.pallas_call(
        _main_kernel,
        grid=(_B_TC // _BT_MAIN,),
        in_specs=[
            pl.BlockSpec((_BT_MAIN, IN), lambda i: (i, 0)),
            pl.BlockSpec((1, IN), lambda i: (0, 0)),
            pl.BlockSpec((_N_BINS, _FPAD, _HID), lambda i: (0, 0, 0)),
            pl.BlockSpec((1, _HID), lambda i: (0, 0)),
            pl.BlockSpec((_HID, _OUT), lambda i: (0, 0)),
            pl.BlockSpec((1, _OUT), lambda i: (0, 0)),
        ],
        out_specs=pl.BlockSpec((_BT_MAIN, _OUT), lambda i: (i, 0)),
        out_shape=jax.ShapeDtypeStruct((B, _OUT), jnp.float32),
        scratch_shapes=[
            pltpu.VMEM((_N_BINS // 2, 2 * _FPAD, _HID), jnp.bfloat16)],
    )(X, cmax, t2, b1.reshape(1, -1), W2.T, b2.reshape(1, -1))

    t3 = jnp.transpose(t2, (1, 0, 2))[:_IN_DIM].reshape(
        _IN_DIM * _N_BINS, _HID)
    t3 = jnp.pad(t3, ((0, 0), (0, _RSTRIDE - _HID))).reshape(-1)
    t3 = jnp.pad(t3, (0, _TWORDS - t3.shape[0]))
    h_pre = _sc_call(codes.reshape(-1), t3).reshape(_B_SC, _HID)

    out = pl.pallas_call(
        _tail_kernel,
        grid=(1,),
        in_specs=[
            pl.BlockSpec((_B_SC, _OUT), lambda i: (B // _B_SC - 1, 0)),
            pl.BlockSpec((_B_SC, _HID), lambda i: (0, 0)),
            pl.BlockSpec((1, _HID), lambda i: (0, 0)),
            pl.BlockSpec((_HID, _OUT), lambda i: (0, 0)),
            pl.BlockSpec((1, _OUT), lambda i: (0, 0)),
        ],
        out_specs=pl.BlockSpec((_B_SC, _OUT), lambda i: (B // _B_SC - 1, 0)),
        out_shape=jax.ShapeDtypeStruct((B, _OUT), jnp.float32),
        input_output_aliases={0: 0},
    )(out_buf, h_pre, b1.reshape(1, -1), W2.T, b2.reshape(1, -1))
    return out


# speedup vs baseline: 1.3020x; 1.3020x over previous
import functools

import jax
import jax.numpy as jnp
from jax import lax
from jax.experimental import pallas as pl
from jax.experimental.pallas import tpu as pltpu
from jax.experimental.pallas import tpu_sc as plsc

_B = 16384
_IN_DIM = 100
_N_BINS = 10
_EMB = 16
_HID = 64
_OUT = 64
_FPAD = 128
_RSTRIDE = 65
_TWORDS = 65536
_BT = 4096

_B_SC = 1024
_B_TC = _B - _B_SC
_BT_MAIN = 3072
_NW = 32
_BW = _B_SC // _NW
_CS = 32


def _prep_kernel(xt_ref, embp_ref, w1s_ref, codes_ref, t2_ref, cmax_ref,
                 cmax_scr):
    i = pl.program_id(0)
    G = pl.num_programs(0) - 1

    @pl.when(i < G)
    def _colmax_phase():
        part = jnp.max(jnp.abs(xt_ref[...]), axis=1, keepdims=True)

        @pl.when(i == 0)
        def _():
            cmax_scr[...] = part

        @pl.when(i > 0)
        def _():
            cmax_scr[...] = jnp.maximum(cmax_scr[...], part)

    @pl.when(i == G)
    def _fold_codes_phase():
        acc = embp_ref[:, :, 0:1] * w1s_ref[0]
        for d in range(1, _EMB):
            acc = acc + embp_ref[:, :, d:d + 1] * w1s_ref[d]
        t2_ref[...] = acc
        cmax_ref[...] = cmax_scr[...]
        xt = xt_ref[:, _BT - _B_SC:]
        d = cmax_scr[...]
        bins = jnp.clip(xt / d * (_N_BINS / 2.0) + _N_BINS / 2.0,
                        0.0, _N_BINS - 1).astype(jnp.int32)
        f_iota = jax.lax.broadcasted_iota(jnp.int32, xt.shape, 0)
        codes_ref[...] = (f_iota * _N_BINS + bins) * _RSTRIDE


def _main_kernel(xt_ref, cmax_ref, t2_ref, b1_ref, w2_ref, b2_ref, o_ref,
                 t2bf_scr):
    i = pl.program_id(0)

    @pl.when(i == 0)
    def _():
        t2bf_scr[...] = t2_ref[...].astype(jnp.bfloat16).reshape(
            _N_BINS // 2, 2 * _FPAD, _HID)

    xt = xt_ref[...]
    d = cmax_ref[...]
    bins = jnp.clip(xt / d * (_N_BINS / 2.0) + _N_BINS / 2.0,
                    0.0, _N_BINS - 1).astype(jnp.int32)
    pad = jnp.full((_FPAD - _IN_DIM, xt.shape[1]), -1, jnp.int32)
    binp = jnp.concatenate([bins, pad], axis=0).astype(jnp.bfloat16)
    bin2 = jnp.concatenate([binp, binp], axis=0)
    lane2 = jax.lax.broadcasted_iota(jnp.int32, (2 * _FPAD, 1), 0)
    off = (lane2 >= _FPAD).astype(jnp.bfloat16)
    ht = None
    for q in range(_N_BINS // 2):
        nvec = off + jnp.bfloat16(2 * q)
        mask = (bin2 == nvec).astype(jnp.bfloat16)
        dq = jax.lax.dot_general(
            t2bf_scr[q], mask, (((0,), (0,)), ((), ())),
            preferred_element_type=jnp.float32)
        ht = dq if ht is None else ht + dq
    ht = jnp.maximum(ht + b1_ref[...], 0.0)
    out = jax.lax.dot_general(
        w2_ref[...], ht, (((1,), (0,)), ((), ())),
        preferred_element_type=jnp.float32)
    o_ref[...] = out + b2_ref[...]


@functools.cache
def _build_sc_lookup():
    return functools.partial(
        pl.kernel,
        mesh=plsc.VectorSubcoreMesh(core_axis_name="c", subcore_axis_name="s"),
        compiler_params=pltpu.CompilerParams(needs_layout_passes=False),
        out_type=jax.ShapeDtypeStruct((_B_SC * _HID,), jnp.float32),
        scratch_types=[
            pltpu.VMEM((_CS * _IN_DIM,), jnp.int32),
            pltpu.VMEM((_TWORDS,), jnp.float32),
            pltpu.VMEM((_CS * _HID,), jnp.float32),
        ],
    )(_sc_lookup_body)


def _sc_call(codes_flat, table_flat):
    return _build_sc_lookup()(codes_flat, table_flat)


def _sc_lookup_body(codes_hbm, table_hbm, out_hbm, codes_v, table_v, h_v):
    wid = lax.axis_index("s") * 2 + lax.axis_index("c")
    lane = lax.iota(jnp.int32, 16)
    lane_c = lane * _IN_DIM
    lane_h = lane * _HID
    pltpu.sync_copy(table_hbm, table_v)

    def chunk_body(c, carry):
        row0 = wid * _BW + c * _CS
        pltpu.sync_copy(codes_hbm.at[pl.ds(row0 * _IN_DIM, _CS * _IN_DIM)],
                        codes_v)

        def g_body(g, carry2):
            for hhc in range(4):
                def f_body(f, accs):
                    bases = plsc.load_gather(
                        codes_v, [lane_c + (g * (16 * _IN_DIM) + f)])
                    return tuple(
                        accs[p] + plsc.load_gather(
                            table_v, [bases + (hhc * 16 + p)])
                        for p in range(16))

                accs = lax.fori_loop(
                    0, _IN_DIM, f_body,
                    tuple(jnp.zeros((16,), jnp.float32) for _ in range(16)))
                for p in range(16):
                    plsc.store_scatter(
                        h_v, [lane_h + (g * (16 * _HID) + hhc * 16 + p)],
                        accs[p])
            return carry2

        lax.fori_loop(0, _CS // 16, g_body, 0)
        pltpu.sync_copy(h_v, out_hbm.at[pl.ds(row0 * _HID, _CS * _HID)])
        return carry

    lax.fori_loop(0, _BW // _CS, chunk_body, 0)


def _tail_kernel(oalias_ref, h_ref, b1_ref, w2_ref, b2_ref, o_ref):
    h = jnp.maximum(h_ref[...] + b1_ref[...], 0.0)
    out = jax.lax.dot_general(
        w2_ref[...], h, (((1,), (1,)), ((), ())),
        preferred_element_type=jnp.float32)
    o_ref[...] = out + b2_ref[...]


def kernel(X, emb, W1, b1, W2, b2):
    B, IN = X.shape
    XT = X.T
    G = B // _BT

    embp = jnp.pad(jnp.transpose(emb, (1, 0, 2)),
                   ((0, 0), (0, _FPAD - _IN_DIM), (0, 0)))
    w1s = jnp.pad(W1.T.reshape(_IN_DIM, _EMB, _HID).transpose(1, 0, 2),
                  ((0, 0), (0, _FPAD - _IN_DIM), (0, 0)))

    codes_t, t2, cmax = pl.pallas_call(
        _prep_kernel,
        grid=(G + 1,),
        in_specs=[
            pl.BlockSpec((IN, _BT), lambda i: (0, jnp.minimum(i, 3))),
            pl.BlockSpec((_N_BINS, _FPAD, _EMB), lambda i: (0, 0, 0)),
            pl.BlockSpec((_EMB, _FPAD, _HID), lambda i: (0, 0, 0)),
        ],
        out_specs=[
            pl.BlockSpec((IN, _B_SC), lambda i: (0, 0)),
            pl.BlockSpec((_N_BINS, _FPAD, _HID), lambda i: (0, 0, 0)),
            pl.BlockSpec((IN, 1), lambda i: (0, 0)),
        ],
        out_shape=[
            jax.ShapeDtypeStruct((IN, _B_SC), jnp.int32),
            jax.ShapeDtypeStruct((_N_BINS, _FPAD, _HID), jnp.float32),
            jax.ShapeDtypeStruct((IN, 1), jnp.float32),
        ],
        scratch_shapes=[pltpu.VMEM((IN, 1), jnp.float32)],
    )(XT, embp, w1s)

    out_t_buf = pl.pallas_call(
        _main_kernel,
        grid=(_B_TC // _BT_MAIN,),
        in_specs=[
            pl.BlockSpec((IN, _BT_MAIN), lambda i: (0, i)),
            pl.BlockSpec((IN, 1), lambda i: (0, 0)),
            pl.BlockSpec((_N_BINS, _FPAD, _HID), lambda i: (0, 0, 0)),
            pl.BlockSpec((_HID, 1), lambda i: (0, 0)),
            pl.BlockSpec((_OUT, _HID), lambda i: (0, 0)),
            pl.BlockSpec((_OUT, 1), lambda i: (0, 0)),
        ],
        out_specs=pl.BlockSpec((_OUT, _BT_MAIN), lambda i: (0, i)),
        out_shape=jax.ShapeDtypeStruct((_OUT, B), jnp.float32),
        scratch_shapes=[
            pltpu.VMEM((_N_BINS // 2, 2 * _FPAD, _HID), jnp.bfloat16)],
    )(XT, cmax, t2, b1.reshape(-1, 1), W2, b2.reshape(-1, 1))

    t3 = jnp.transpose(t2, (1, 0, 2))[:_IN_DIM].reshape(
        _IN_DIM * _N_BINS, _HID)
    t3 = jnp.pad(t3, ((0, 0), (0, _RSTRIDE - _HID))).reshape(-1)
    t3 = jnp.pad(t3, (0, _TWORDS - t3.shape[0]))
    h_pre = _sc_call(codes_t.T.reshape(-1), t3).reshape(_B_SC, _HID)

    out_t = pl.pallas_call(
        _tail_kernel,
        grid=(1,),
        in_specs=[
            pl.BlockSpec((_OUT, _B_SC), lambda i: (0, B // _B_SC - 1)),
            pl.BlockSpec((_B_SC, _HID), lambda i: (0, 0)),
            pl.BlockSpec((1, _HID), lambda i: (0, 0)),
            pl.BlockSpec((_OUT, _HID), lambda i: (0, 0)),
            pl.BlockSpec((_OUT, 1), lambda i: (0, 0)),
        ],
        out_specs=pl.BlockSpec((_OUT, _B_SC), lambda i: (0, B // _B_SC - 1)),
        out_shape=jax.ShapeDtypeStruct((_OUT, B), jnp.float32),
        input_output_aliases={0: 0},
    )(out_t_buf, h_pre, b1.reshape(1, -1), W2, b2.reshape(-1, 1))
    return out_t.T
